# Initial kernel scaffold; baseline (speedup 1.0000x reference)
import jax
import jax.numpy as jnp
from jax.experimental import pallas as pl


def kernel(data, indices):
    raise NotImplementedError("write your pallas kernel here")


# speedup vs baseline: 1.2154x; 1.2154x over previous
import jax
import jax.numpy as jnp
from jax import lax
from jax.experimental import pallas as pl
from jax.experimental.pallas import tpu as pltpu
from jax.experimental.pallas import tpu_sc as plsc

R, V, B = 64, 1_000_000, 16384
NC, NS = 2, 16
RPC = R // NC
SEG = B // NS


def _body(data_hbm, idx_hbm, out_hbm, idx_v, dst_v, row_sh, rsem, gsem, osem):
    cid = lax.axis_index("c")
    sid = lax.axis_index("s")
    pltpu.sync_copy(idx_hbm.at[pl.ds(sid * SEG, SEG)], idx_v)

    for k in range(RPC):
        r = cid * RPC + k

        @pl.when(sid == 0)
        def _stage_row():
            pltpu.async_copy(data_hbm.at[r], row_sh, rsem).wait()

        plsc.subcore_barrier()
        pltpu.async_copy(row_sh.at[idx_v], dst_v, gsem).wait()
        pltpu.async_copy(dst_v, out_hbm.at[r].at[pl.ds(sid * SEG, SEG)],
                         osem).wait()
        plsc.subcore_barrier()


def kernel(data, indices):
    mesh = plsc.VectorSubcoreMesh(core_axis_name="c", subcore_axis_name="s")
    gather = pl.kernel(
        _body,
        mesh=mesh,
        out_type=jax.ShapeDtypeStruct((R, B), jnp.float32),
        scratch_types=[
            pltpu.VMEM((SEG,), jnp.int32),
            pltpu.VMEM((SEG,), jnp.float32),
            pltpu.VMEM_SHARED((V,), jnp.float32),
            pltpu.SemaphoreType.DMA,
            pltpu.SemaphoreType.DMA,
            pltpu.SemaphoreType.DMA,
        ],
    )
    return gather(data, indices)
